# baseline (speedup 1.0000x reference)
import functools

import jax
import jax.numpy as jnp
from jax import lax
from jax.experimental import pallas as pl
from jax.experimental.pallas import tpu as pltpu
from jax.experimental.pallas import tpu_sc as plsc

_NW = 32


def _interleave(x_buf, obuf, L, D):
    for l in range(L):
        for m in range(0, D, 16):
            obuf[pl.ds(2 * D * l + m, 16)] = x_buf[pl.ds(D * l + m, 16)]


def kernel(x, position_table):
    B, L, D = x.shape
    XW = L * D
    OW = L * 2 * D
    pos = position_table[:L]
    tmpl = jnp.concatenate(
        [jnp.zeros((L, D), pos.dtype), pos], axis=-1).reshape(OW)
    x2 = x.reshape(B, XW)
    rows = B // _NW

    @functools.partial(
        pl.kernel,
        mesh=plsc.VectorSubcoreMesh(core_axis_name="c", subcore_axis_name="s"),
        out_type=jax.ShapeDtypeStruct((B, OW), x.dtype),
        scratch_types=[
            pltpu.VMEM((XW,), jnp.float32),
            pltpu.VMEM((XW,), jnp.float32),
            pltpu.VMEM((OW,), jnp.float32),
            pltpu.VMEM((OW,), jnp.float32),
            pltpu.SemaphoreType.DMA((2,)),
            pltpu.SemaphoreType.DMA((2,)),
        ],
    )
    def run(x_hbm, tmpl_hbm, out_hbm, xb0, xb1, ob0, ob1, in_sems, out_sems):
        c = lax.axis_index("c")
        s = lax.axis_index("s")
        base = (s * 2 + c) * rows

        pltpu.sync_copy(tmpl_hbm, ob0)
        pltpu.sync_copy(tmpl_hbm, ob1)
        pltpu.async_copy(x_hbm.at[base], xb0, in_sems.at[0])
        pltpu.async_copy(x_hbm.at[base + 1], xb1, in_sems.at[1])

        def half(r, xb, ob, in_sem, out_sem):
            pltpu.make_async_copy(x_hbm.at[base + r], xb, in_sem).wait()

            @pl.when(r >= 2)
            def _():
                pltpu.make_async_copy(
                    ob, out_hbm.at[base + r - 2], out_sem).wait()

            _interleave(xb, ob, L, D)
            pltpu.async_copy(ob, out_hbm.at[base + r], out_sem)

            @pl.when(r + 2 < rows)
            def _():
                pltpu.async_copy(x_hbm.at[base + r + 2], xb, in_sem)

        def step(k, carry):
            r = 2 * k
            half(r, xb0, ob0, in_sems.at[0], out_sems.at[0])
            half(r + 1, xb1, ob1, in_sems.at[1], out_sems.at[1])
            return carry

        lax.fori_loop(0, rows // 2, step, 0)
        pltpu.make_async_copy(ob0, out_hbm.at[base + rows - 2],
                              out_sems.at[0]).wait()
        pltpu.make_async_copy(ob1, out_hbm.at[base + rows - 1],
                              out_sems.at[1]).wait()

    out = run(x2, tmpl)
    return out.reshape(B, L, 2 * D)


# speedup vs baseline: 1.6387x; 1.6387x over previous
import functools

import jax
import jax.numpy as jnp
from jax import lax
from jax.experimental import pallas as pl
from jax.experimental.pallas import tpu as pltpu
from jax.experimental.pallas import tpu_sc as plsc

_NW = 32


def _interleave(x_buf, obuf, L, D):
    for l in range(L):
        for m in range(0, D, 16):
            obuf[l, pl.ds(m, 16)] = x_buf[pl.ds(D * l + m, 16)]


def kernel(x, position_table):
    B, L, D = x.shape
    XW = L * D
    pos = position_table[:L]
    tmpl = jnp.concatenate([jnp.zeros((L, D), pos.dtype), pos], axis=-1)
    x2 = x.reshape(B, XW)
    rows = B // _NW

    @functools.partial(
        pl.kernel,
        mesh=plsc.VectorSubcoreMesh(core_axis_name="c", subcore_axis_name="s"),
        out_type=jax.ShapeDtypeStruct((B, L, 2 * D), x.dtype),
        scratch_types=[
            pltpu.VMEM((XW,), jnp.float32),
            pltpu.VMEM((XW,), jnp.float32),
            pltpu.VMEM((L, 2 * D), jnp.float32),
            pltpu.VMEM((L, 2 * D), jnp.float32),
            pltpu.SemaphoreType.DMA((2,)),
            pltpu.SemaphoreType.DMA((2,)),
        ],
    )
    def run(x_hbm, tmpl_hbm, out_hbm, xb0, xb1, ob0, ob1, in_sems, out_sems):
        c = lax.axis_index("c")
        s = lax.axis_index("s")
        base = (s * 2 + c) * rows

        pltpu.sync_copy(tmpl_hbm, ob0)
        pltpu.sync_copy(tmpl_hbm, ob1)
        pltpu.async_copy(x_hbm.at[base], xb0, in_sems.at[0])
        pltpu.async_copy(x_hbm.at[base + 1], xb1, in_sems.at[1])

        def half(r, xb, ob, in_sem, out_sem):
            pltpu.make_async_copy(x_hbm.at[base + r], xb, in_sem).wait()

            @pl.when(r >= 2)
            def _():
                pltpu.make_async_copy(
                    ob, out_hbm.at[base + r - 2], out_sem).wait()

            _interleave(xb, ob, L, D)
            pltpu.async_copy(ob, out_hbm.at[base + r], out_sem)

            @pl.when(r + 2 < rows)
            def _():
                pltpu.async_copy(x_hbm.at[base + r + 2], xb, in_sem)

        def step(k, carry):
            r = 2 * k
            half(r, xb0, ob0, in_sems.at[0], out_sems.at[0])
            half(r + 1, xb1, ob1, in_sems.at[1], out_sems.at[1])
            return carry

        lax.fori_loop(0, rows // 2, step, 0)
        pltpu.make_async_copy(ob0, out_hbm.at[base + rows - 2],
                              out_sems.at[0]).wait()
        pltpu.make_async_copy(ob1, out_hbm.at[base + rows - 1],
                              out_sems.at[1]).wait()

    return run(x2, tmpl)
